# baseline (speedup 1.0000x reference)
import functools

import jax
import jax.numpy as jnp
from jax import lax
from jax.experimental import pallas as pl
from jax.experimental.pallas import tpu as pltpu
from jax.experimental.pallas import tpu_sc as plsc

NSC = 2
NTILE = 16
LANES = 16

M = 1000000
D = 64
B = 16384
UPD = B * D
R = 8192
RSHIFT = 13
CHUNK = R * D
CH = (M + R - 1) // R
TAILR = M - (M // R) * R
P = 64
SLAB = CHUNK // NTILE
TAIL_SLAB = TAILR * D // NTILE
TRASH = CHUNK
RS = 16384
NR = UPD // RS
NRT = NR // NTILE
CHP = 128
BPAD = 512
BLK = 512
MSZ = 2 * NR * CHP

_mesh = plsc.VectorSubcoreMesh(
    core_axis_name="c", subcore_axis_name="s",
    num_cores=NSC, num_subcores=NTILE)


def _scatter_add_dma(val_row, slab_ref, tgt_row):
    pltpu.sync_copy(val_row, slab_ref.at[tgt_row], add=True)


@functools.partial(
    pl.kernel, mesh=_mesh,
    out_type=[
        jax.ShapeDtypeStruct((M * D,), jnp.float32),
        jax.ShapeDtypeStruct((NSC * UPD + BPAD,), jnp.int32),
        jax.ShapeDtypeStruct((NSC * UPD + BPAD,), jnp.float32),
        jax.ShapeDtypeStruct((MSZ,), jnp.int32),
    ],
    scratch_types=[
        pltpu.VMEM((RS,), jnp.int32),
        pltpu.VMEM((RS,), jnp.float32),
        pltpu.VMEM((RS,), jnp.int32),
        pltpu.VMEM((RS,), jnp.float32),
        pltpu.VMEM((CHP * LANES,), jnp.int32),
        pltpu.VMEM((CHP * LANES,), jnp.int32),
        pltpu.VMEM((CHP,), jnp.int32),
        pltpu.VMEM((CHP,), jnp.int32),
        pltpu.VMEM((MSZ,), jnp.int32),
        pltpu.VMEM((4, BLK // 128, 128), jnp.int32),
        pltpu.VMEM((4, BLK // 128, 128), jnp.float32),
        pltpu.VMEM_SHARED((CHUNK + 2048,), jnp.float32),
        pltpu.SemaphoreType.DMA,
        pltpu.SemaphoreType.DMA,
    ],
    compiler_params=pltpu.CompilerParams(needs_layout_passes=False),
)
def _scatter_add(x_hbm, idx_hbm, src_hbm, out_hbm, tgt_hbm, val_hbm,
                 meta_hbm,
                 idxr, srcr, tgts, vals, hist, curs, sstart, slen,
                 metav, tb, vb, slab_sh, sem, sem2):
    cid = lax.axis_index("c")
    sid = lax.axis_index("s")
    iota = lax.iota(jnp.int32, LANES)
    zeros = iota * 0
    binbase = cid * P
    disc = CHP - 1 - cid

    def region(rr, _):
        r = sid * NRT + rr
        pltpu.sync_copy(idx_hbm.at[pl.ds(r * RS, RS)], idxr)
        pltpu.sync_copy(src_hbm.at[pl.ds(r * RS, RS)], srcr)

        for c in range(CHP):
            hist[pl.ds(c * LANES, LANES)] = zeros

        def hist_step(k, _):
            v = idxr[pl.ds(k * LANES, LANES)]
            ch = v >> RSHIFT
            cb = jnp.where((ch >= binbase) & (ch < binbase + P), ch, disc)
            slot = (cb << 4) + iota
            h = plsc.load_gather(hist, [slot])
            plsc.store_scatter(hist, [slot], h + 1)
            return 0
        lax.fori_loop(0, RS // LANES, hist_step, 0)

        run = jnp.int32(0)
        sv = [zeros] * (CHP // LANES)
        lv = [zeros] * (CHP // LANES)
        for c in range(CHP):
            v = hist[pl.ds(c * LANES, LANES)]
            ex = plsc.cumsum(v) - v
            curs[pl.ds(c * LANES, LANES)] = ex + run
            tot = jnp.sum(v)
            g, l = c // LANES, c % LANES
            sv[g] = jnp.where(iota == l, run + (2 * r + cid) * RS, sv[g])
            lv[g] = jnp.where(iota == l, tot, lv[g])
            run = run + tot
        for g in range(CHP // LANES):
            sstart[pl.ds(g * LANES, LANES)] = sv[g]
            slen[pl.ds(g * LANES, LANES)] = lv[g]
        H = CHP // 2

        @pl.when(cid == 0)
        def _():
            mo = pl.multiple_of(r * CHP, 8)
            pltpu.sync_copy(sstart.at[pl.ds(0, H)],
                            meta_hbm.at[pl.ds(mo, H)])
            mo2 = pl.multiple_of(NR * CHP + r * CHP, 8)
            pltpu.sync_copy(slen.at[pl.ds(0, H)],
                            meta_hbm.at[pl.ds(mo2, H)])

        @pl.when(cid == 1)
        def _():
            mo = pl.multiple_of(r * CHP + H, 8)
            pltpu.sync_copy(sstart.at[pl.ds(H, H)],
                            meta_hbm.at[pl.ds(mo, H)])
            mo2 = pl.multiple_of(NR * CHP + r * CHP + H, 8)
            pltpu.sync_copy(slen.at[pl.ds(H, H)],
                            meta_hbm.at[pl.ds(mo2, H)])

        def place_step(k, _):
            v = idxr[pl.ds(k * LANES, LANES)]
            s = srcr[pl.ds(k * LANES, LANES)]
            ch = v >> RSHIFT
            cb = jnp.where((ch >= binbase) & (ch < binbase + P), ch, disc)
            slot = (cb << 4) + iota
            pos = plsc.load_gather(curs, [slot])
            plsc.store_scatter(curs, [slot], pos + 1)
            colv = (k * LANES + iota) & (D - 1)
            lf = ((v & (R - 1)) << 6) | colv
            plsc.store_scatter(tgts, [pos], lf)
            plsc.store_scatter(vals, [pos], s)
            return 0
        lax.fori_loop(0, RS // LANES, place_step, 0)

        fo = pl.multiple_of((2 * r + cid) * RS, 8)
        pltpu.sync_copy(tgts, tgt_hbm.at[pl.ds(fo, RS)])
        pltpu.sync_copy(vals, val_hbm.at[pl.ds(fo, RS)])
        return 0

    lax.fori_loop(0, NRT, region, 0)
    plsc.subcore_barrier()

    pltpu.sync_copy(meta_hbm, metav)

    def chunk_round(p, _):
        chunk = binbase + p
        live = chunk < CH
        is_tail = chunk == CH - 1

        @pl.when(live & jnp.logical_not(is_tail))
        def _():
            off = chunk * CHUNK + sid * SLAB
            pltpu.sync_copy(x_hbm.at[pl.ds(off, SLAB)],
                            slab_sh.at[pl.ds(sid * SLAB, SLAB)])

        @pl.when(is_tail)
        def _():
            off = chunk * CHUNK + sid * TAIL_SLAB
            pltpu.sync_copy(x_hbm.at[pl.ds(off, TAIL_SLAB)],
                            slab_sh.at[pl.ds(sid * TAIL_SLAB, TAIL_SLAB)])
        plsc.subcore_barrier()

        @pl.when(live)
        def _():
            midx = (sid * 4 + (iota & 3)) * CHP + chunk \
                + jnp.where((iota >= 4) & (iota < 8), NR * CHP, 0)
            midx = jnp.where(iota < 8, midx, 0)
            mv = plsc.load_gather(metav, [midx])

            g0s, exts, tots = [], [], []
            for rgn in range(4):
                g0 = jnp.sum(jnp.where(iota == rgn, mv, 0))
                ln = jnp.sum(jnp.where(iota == rgn + 4, mv, 0))
                al = g0 & ~7
                g0s.append(al)
                exts.append(g0 - al)
                tots.append((g0 - al) + ln)

            def fire_loads(rgn):
                ds_ = []
                for j in range(BLK // 128):
                    po = pl.multiple_of(g0s[rgn] + j * 128, 8)
                    ds_.append(pltpu.async_copy(
                        tgt_hbm.at[pl.ds(po, 128)], tb.at[rgn, j], sem))
                    ds_.append(pltpu.async_copy(
                        val_hbm.at[pl.ds(po, 128)], vb.at[rgn, j], sem))
                return ds_

            ld = {0: fire_loads(0), 1: fire_loads(1)}
            st = []
            for rgn in range(4):
                for d in ld[rgn]:
                    d.wait()
                for j in range(BLK // 128):
                    for gg in range(8):
                        q = j * 128 + gg * LANES + iota
                        ok = (q >= exts[rgn]) & (q < tots[rgn])
                        u = tb[rgn, j, pl.ds(gg * LANES, LANES)]
                        tr = TRASH + ((rgn * BLK + j * 128 + gg * LANES)
                                      & 2047) + iota
                        tb[rgn, j, pl.ds(gg * LANES, LANES)] = \
                            jnp.where(ok, u, tr)
                for j in range(BLK // 128):
                    st.append(pltpu.async_copy(
                        vb.at[rgn, j], slab_sh.at[tb.at[rgn, j]], sem2,
                        add=True))
                if rgn + 2 < 4:
                    ld[rgn + 2] = fire_loads(rgn + 2)
            for d in st:
                d.wait()

            for rgn in range(4):
                def block(b, _):
                    boff = g0s[rgn] + b * BLK
                    for j in range(BLK // 128):
                        po = pl.multiple_of(boff + j * 128, 8)
                        pltpu.sync_copy(
                            tgt_hbm.at[pl.ds(po, 128)], tb.at[0, j])
                        pltpu.sync_copy(
                            val_hbm.at[pl.ds(po, 128)], vb.at[0, j])
                    for j in range(BLK // 128):
                        for gg in range(8):
                            q = b * BLK + j * 128 + gg * LANES + iota
                            ok = (q >= exts[rgn]) & (q < tots[rgn])
                            u = tb[0, j, pl.ds(gg * LANES, LANES)]
                            tr = TRASH + ((j * 128 + gg * LANES) & 2047) \
                                + iota
                            tb[0, j, pl.ds(gg * LANES, LANES)] = \
                                jnp.where(ok, u, tr)
                    for j in range(BLK // 128):
                        _scatter_add_dma(vb.at[0, j], slab_sh, tb.at[0, j])
                    return 0
                nb = (tots[rgn] + BLK - 1) >> 9
                lax.fori_loop(1, nb, block, 0)
        plsc.subcore_barrier()

        @pl.when(live & jnp.logical_not(is_tail))
        def _():
            off = chunk * CHUNK + sid * SLAB
            pltpu.sync_copy(slab_sh.at[pl.ds(sid * SLAB, SLAB)],
                            out_hbm.at[pl.ds(off, SLAB)])

        @pl.when(is_tail)
        def _():
            off = chunk * CHUNK + sid * TAIL_SLAB
            pltpu.sync_copy(slab_sh.at[pl.ds(sid * TAIL_SLAB, TAIL_SLAB)],
                            out_hbm.at[pl.ds(off, TAIL_SLAB)])
        return 0

    lax.fori_loop(0, P, chunk_round, 0)


def kernel(x, idx, src):
    xf = x.reshape(M * D)
    idxf = idx.reshape(UPD)
    srcf = src.reshape(UPD)
    out, _, _, _ = _scatter_add(xf, idxf, srcf)
    return out.reshape(M, D)


# speedup vs baseline: 1.0862x; 1.0862x over previous
import functools

import jax
import jax.numpy as jnp
from jax import lax
from jax.experimental import pallas as pl
from jax.experimental.pallas import tpu as pltpu
from jax.experimental.pallas import tpu_sc as plsc

NSC = 2
NTILE = 16
LANES = 16

M = 1000000
D = 64
B = 16384
UPD = B * D
R = 8192
RSHIFT = 13
CHUNK = R * D
CH = (M + R - 1) // R
FULLC = M // R
TAILR = M - FULLC * R
P = (CH + NSC - 1) // NSC
SLAB = CHUNK // NTILE
TAIL_ELEMS = TAILR * D
TAIL_SLAB = TAIL_ELEMS // NTILE
TRASH = CHUNK
RS = 16384
NR = UPD // RS
NRT = NR // (NSC * NTILE)
CHP = 128
BPAD = 512
BLK = 512

_mesh = plsc.VectorSubcoreMesh(
    core_axis_name="c", subcore_axis_name="s",
    num_cores=NSC, num_subcores=NTILE)


def _scatter_add_dma(val_row, slab_ref, tgt_row):
    pltpu.sync_copy(val_row, slab_ref.at[tgt_row], add=True)


@functools.partial(
    pl.kernel, mesh=_mesh,
    out_type=[
        jax.ShapeDtypeStruct((UPD + BPAD,), jnp.int32),
        jax.ShapeDtypeStruct((UPD + BPAD,), jnp.float32),
        jax.ShapeDtypeStruct((2, NR, CHP), jnp.int32),
    ],
    scratch_types=[
        pltpu.VMEM((RS,), jnp.int32),
        pltpu.VMEM((RS,), jnp.float32),
        pltpu.VMEM((RS,), jnp.int32),
        pltpu.VMEM((RS,), jnp.float32),
        pltpu.VMEM((CHP * LANES,), jnp.int32),
        pltpu.VMEM((CHP * LANES,), jnp.int32),
        pltpu.VMEM((CHP,), jnp.int32),
        pltpu.VMEM((CHP,), jnp.int32),
    ],
    compiler_params=pltpu.CompilerParams(needs_layout_passes=False),
)
def _bucketize(idx_hbm, src_hbm, tgt_hbm, val_hbm, meta_hbm,
               idxr, srcr, tgts, vals, hist, curs, sstart, slen):
    cid = lax.axis_index("c")
    sid = lax.axis_index("s")
    wid = sid * NSC + cid
    iota = lax.iota(jnp.int32, LANES)
    zeros = iota * 0

    for rr in range(NRT):
        r = wid * NRT + rr
        pltpu.sync_copy(idx_hbm.at[pl.ds(r * RS, RS)], idxr)
        pltpu.sync_copy(src_hbm.at[pl.ds(r * RS, RS)], srcr)

        for c in range(CHP):
            hist[pl.ds(c * LANES, LANES)] = zeros

        def hist_step(k, _):
            v = idxr[pl.ds(k * LANES, LANES)]
            slot = ((v >> RSHIFT) << 4) + iota
            h = plsc.load_gather(hist, [slot])
            plsc.store_scatter(hist, [slot], h + 1)
            return 0
        lax.fori_loop(0, RS // LANES, hist_step, 0)

        run = jnp.int32(0)
        sv = [zeros] * (CHP // LANES)
        lv = [zeros] * (CHP // LANES)
        for c in range(CHP):
            v = hist[pl.ds(c * LANES, LANES)]
            ex = plsc.cumsum(v) - v
            curs[pl.ds(c * LANES, LANES)] = ex + run
            tot = jnp.sum(v)
            g, l = c // LANES, c % LANES
            sv[g] = jnp.where(iota == l, run + r * RS, sv[g])
            lv[g] = jnp.where(iota == l, tot, lv[g])
            run = run + tot
        for g in range(CHP // LANES):
            sstart[pl.ds(g * LANES, LANES)] = sv[g]
            slen[pl.ds(g * LANES, LANES)] = lv[g]
        pltpu.sync_copy(sstart, meta_hbm.at[0, r])
        pltpu.sync_copy(slen, meta_hbm.at[1, r])

        def place_step(k, _):
            v = idxr[pl.ds(k * LANES, LANES)]
            s = srcr[pl.ds(k * LANES, LANES)]
            slot = ((v >> RSHIFT) << 4) + iota
            pos = plsc.load_gather(curs, [slot])
            plsc.store_scatter(curs, [slot], pos + 1)
            colv = (k * LANES + iota) & (D - 1)
            lf = ((v & (R - 1)) << 6) | colv
            plsc.store_scatter(tgts, [pos], lf)
            plsc.store_scatter(vals, [pos], s)
            return 0
        lax.fori_loop(0, RS // LANES, place_step, 0)

        pltpu.sync_copy(tgts, tgt_hbm.at[pl.ds(r * RS, RS)])
        pltpu.sync_copy(vals, val_hbm.at[pl.ds(r * RS, RS)])


@functools.partial(
    pl.kernel, mesh=_mesh,
    out_type=jax.ShapeDtypeStruct((M * D,), jnp.float32),
    scratch_types=[
        pltpu.VMEM((2 * NR * CHP,), jnp.int32),
        pltpu.VMEM((4, BLK // 128, 128), jnp.int32),
        pltpu.VMEM((4, BLK // 128, 128), jnp.float32),
        pltpu.VMEM_SHARED((CHUNK + 2048,), jnp.float32),
        pltpu.SemaphoreType.DMA,
        pltpu.SemaphoreType.DMA,
    ],
    compiler_params=pltpu.CompilerParams(needs_layout_passes=False),
)
def _apply(x_hbm, tgt_hbm, val_hbm, meta_hbm, out_hbm,
           metav, tb, vb, slab_sh, sem, sem2):
    cid = lax.axis_index("c")
    sid = lax.axis_index("s")
    iota = lax.iota(jnp.int32, LANES)

    pltpu.sync_copy(meta_hbm, metav)

    def chunk_round(p, _):
        chunk = cid * P + p
        live = chunk < CH
        is_tail = chunk == CH - 1

        @pl.when(live & jnp.logical_not(is_tail))
        def _():
            off = chunk * CHUNK + sid * SLAB
            pltpu.sync_copy(x_hbm.at[pl.ds(off, SLAB)],
                            slab_sh.at[pl.ds(sid * SLAB, SLAB)])

        @pl.when(is_tail)
        def _():
            off = chunk * CHUNK + sid * TAIL_SLAB
            pltpu.sync_copy(x_hbm.at[pl.ds(off, TAIL_SLAB)],
                            slab_sh.at[pl.ds(sid * TAIL_SLAB, TAIL_SLAB)])
        plsc.subcore_barrier()

        @pl.when(live)
        def _():
            midx = (sid * 4 + (iota & 3)) * CHP + chunk \
                + jnp.where((iota >= 4) & (iota < 8), NR * CHP, 0)
            midx = jnp.where(iota < 8, midx, 0)
            mv = plsc.load_gather(metav, [midx])

            g0s, exts, tots = [], [], []
            for rgn in range(4):
                g0 = jnp.sum(jnp.where(iota == rgn, mv, 0))
                ln = jnp.sum(jnp.where(iota == rgn + 4, mv, 0))
                al = g0 & ~7
                g0s.append(al)
                exts.append(g0 - al)
                tots.append((g0 - al) + ln)

            def fire_loads(rgn):
                ds_ = []
                for j in range(BLK // 128):
                    po = pl.multiple_of(g0s[rgn] + j * 128, 8)
                    ds_.append(pltpu.async_copy(
                        tgt_hbm.at[pl.ds(po, 128)], tb.at[rgn, j], sem))
                    ds_.append(pltpu.async_copy(
                        val_hbm.at[pl.ds(po, 128)], vb.at[rgn, j], sem))
                return ds_

            ld = {0: fire_loads(0), 1: fire_loads(1)}
            st = []
            for rgn in range(4):
                for d in ld[rgn]:
                    d.wait()
                for j in range(BLK // 128):
                    for gg in range(8):
                        q = j * 128 + gg * LANES + iota
                        ok = (q >= exts[rgn]) & (q < tots[rgn])
                        u = tb[rgn, j, pl.ds(gg * LANES, LANES)]
                        tr = TRASH + ((rgn * BLK + j * 128 + gg * LANES)
                                      & 2047) + iota
                        tb[rgn, j, pl.ds(gg * LANES, LANES)] = \
                            jnp.where(ok, u, tr)
                for j in range(BLK // 128):
                    st.append(pltpu.async_copy(
                        vb.at[rgn, j], slab_sh.at[tb.at[rgn, j]], sem2,
                        add=True))
                if rgn + 2 < 4:
                    ld[rgn + 2] = fire_loads(rgn + 2)
            for d in st:
                d.wait()

            for rgn in range(4):
                def block(b, _):
                    boff = g0s[rgn] + b * BLK
                    for j in range(BLK // 128):
                        po = pl.multiple_of(boff + j * 128, 8)
                        pltpu.sync_copy(
                            tgt_hbm.at[pl.ds(po, 128)], tb.at[0, j])
                        pltpu.sync_copy(
                            val_hbm.at[pl.ds(po, 128)], vb.at[0, j])
                    for j in range(BLK // 128):
                        for gg in range(8):
                            q = b * BLK + j * 128 + gg * LANES + iota
                            ok = (q >= exts[rgn]) & (q < tots[rgn])
                            u = tb[0, j, pl.ds(gg * LANES, LANES)]
                            tr = TRASH + ((j * 128 + gg * LANES) & 2047) \
                                + iota
                            tb[0, j, pl.ds(gg * LANES, LANES)] = \
                                jnp.where(ok, u, tr)
                    for j in range(BLK // 128):
                        _scatter_add_dma(vb.at[0, j], slab_sh, tb.at[0, j])
                    return 0
                nb = (tots[rgn] + BLK - 1) >> 9
                lax.fori_loop(1, nb, block, 0)
        plsc.subcore_barrier()

        @pl.when(live & jnp.logical_not(is_tail))
        def _():
            off = chunk * CHUNK + sid * SLAB
            pltpu.sync_copy(slab_sh.at[pl.ds(sid * SLAB, SLAB)],
                            out_hbm.at[pl.ds(off, SLAB)])

        @pl.when(is_tail)
        def _():
            off = chunk * CHUNK + sid * TAIL_SLAB
            pltpu.sync_copy(slab_sh.at[pl.ds(sid * TAIL_SLAB, TAIL_SLAB)],
                            out_hbm.at[pl.ds(off, TAIL_SLAB)])
        return 0

    lax.fori_loop(0, P, chunk_round, 0)


def kernel(x, idx, src):
    xf = x.reshape(M * D)
    idxf = idx.reshape(UPD)
    srcf = src.reshape(UPD)
    tgtb, valb, meta = _bucketize(idxf, srcf)
    out = _apply(xf, tgtb, valb, meta.reshape(2 * NR * CHP))
    return out.reshape(M, D)
